# Initial kernel scaffold; baseline (speedup 1.0000x reference)
import jax
import jax.numpy as jnp
from jax.experimental import pallas as pl


def kernel(weakly_points, res1_xyz, res1_features, res2_xyz, res2_features, res3_xyz, res3_features, res4_xyz, res4_features, res5_xyz, res5_features, batch_inds, W1, W2, W3, W4, b4):
    raise NotImplementedError("write your pallas kernel here")


# speedup vs baseline: 3.3427x; 3.3427x over previous
import functools

import jax
import jax.numpy as jnp
from jax.experimental import pallas as pl

_INTERPRET = False


def _interp_stage_body(q_ref, b_ref, x_ref, f_ref, o_ref, *, two_n):
    n = two_n // 2
    qb = q_ref.shape[0]
    X = x_ref[...]
    sumx = jnp.sum(X * X, axis=0, keepdims=True)
    qv = q_ref[:, 0:3]
    sumq = jnp.sum(qv * qv, axis=1, keepdims=True)
    Xb = X.astype(jnp.bfloat16).astype(jnp.float32)
    qb16 = qv.astype(jnp.bfloat16).astype(jnp.float32)
    cross = (qb16[:, 0:1] * Xb[0:1, :]
             + qb16[:, 1:2] * Xb[1:2, :]
             + qb16[:, 2:3] * Xb[2:3, :])
    d2 = sumq + sumx - 2.0 * cross
    lane = jax.lax.broadcasted_iota(jnp.int32, (qb, two_n), 1)
    lane_batch = (lane >= n).astype(jnp.int32)
    bq = b_ref[:, 0:1]
    d2 = jnp.where(lane_batch == bq, d2, jnp.inf)

    vals, idxs = [], []
    for k in range(3):
        m = jnp.min(d2, axis=1, keepdims=True)
        im = jnp.min(jnp.where(d2 == m, lane, two_n), axis=1, keepdims=True)
        vals.append(m)
        idxs.append(im)
        if k < 2:
            d2 = jnp.where(lane == im, jnp.inf, d2)

    rec = [1.0 / (jnp.maximum(v, 1e-10) + 1e-8) for v in vals]
    rsum = rec[0] + rec[1] + rec[2]
    w = [r / rsum for r in rec]
    wfull = (jnp.where(lane == idxs[0], w[0], 0.0)
             + jnp.where(lane == idxs[1], w[1], 0.0)
             + jnp.where(lane == idxs[2], w[2], 0.0))
    o_ref[...] = jax.lax.dot_general(
        f_ref[...], wfull,
        dimension_numbers=(((1,), (1,)), ((), ())),
        preferred_element_type=jnp.float32)


def _interp_stage(qpad, bpad, xyzT, feat, qb):
    nq = qpad.shape[0]
    c, two_n = feat.shape
    grid = nq // qb
    body = functools.partial(_interp_stage_body, two_n=two_n)
    body = functools.wraps(_interp_stage_body)(body)
    body.__name__ = f"interp_c{c}_n{two_n}_q{qb}"
    return pl.pallas_call(
        body,
        grid=(grid,),
        in_specs=[
            pl.BlockSpec((qb, 128), lambda i: (i, 0)),
            pl.BlockSpec((qb, 128), lambda i: (i, 0)),
            pl.BlockSpec((3, two_n), lambda i: (0, 0)),
            pl.BlockSpec((c, two_n), lambda i: (0, 0)),
        ],
        out_specs=pl.BlockSpec((c, qb), lambda i: (0, i)),
        out_shape=jax.ShapeDtypeStruct((c, nq), jnp.float32),
        interpret=_INTERPRET,
    )(qpad, bpad, xyzT, feat)


def _mlp_body(x_ref, w1_ref, w2_ref, w3_ref, w4_ref, b4_ref, o_ref):
    x = x_ref[...]
    h = jax.nn.relu(jnp.dot(w1_ref[...], x, preferred_element_type=jnp.float32))
    h = jax.nn.relu(jnp.dot(w2_ref[...], h, preferred_element_type=jnp.float32))
    h = jax.nn.relu(jnp.dot(w3_ref[...], h, preferred_element_type=jnp.float32))
    o_ref[...] = jnp.dot(w4_ref[...], h,
                         preferred_element_type=jnp.float32) + b4_ref[:, 0:1]


def _mlp(xT, W1, W2, W3p, W4p, b4b, qb):
    width, nq = xT.shape
    h1, h2 = W1.shape[0], W2.shape[0]
    h3p, nc = W3p.shape[0], W4p.shape[0]
    grid = nq // qb
    return pl.pallas_call(
        _mlp_body,
        grid=(grid,),
        in_specs=[
            pl.BlockSpec((width, qb), lambda i: (0, i)),
            pl.BlockSpec((h1, width), lambda i: (0, 0)),
            pl.BlockSpec((h2, h1), lambda i: (0, 0)),
            pl.BlockSpec((h3p, h2), lambda i: (0, 0)),
            pl.BlockSpec((nc, h3p), lambda i: (0, 0)),
            pl.BlockSpec((nc, 128), lambda i: (0, 0)),
        ],
        out_specs=pl.BlockSpec((nc, qb), lambda i: (0, i)),
        out_shape=jax.ShapeDtypeStruct((nc, nq), jnp.float32),
        interpret=_INTERPRET,
    )(xT, W1, W2, W3p, W4p, b4b)


def kernel(weakly_points, res1_xyz, res1_features, res2_xyz, res2_features,
           res3_xyz, res3_features, res4_xyz, res4_features, res5_xyz,
           res5_features, batch_inds, W1, W2, W3, W4, b4):
    nq = weakly_points.shape[0]
    qpad = jnp.pad(weakly_points, ((0, 0), (0, 125)))
    bpad = jnp.pad(batch_inds.reshape(-1, 1), ((0, 0), (0, 127)))
    stages = [(res1_xyz, res1_features), (res2_xyz, res2_features),
              (res3_xyz, res3_features), (res4_xyz, res4_features),
              (res5_xyz, res5_features)]
    outs = []
    for (xyz, feat), qb in zip(stages, (128, 256, 512, 512, 512)):
        xyzT = jnp.concatenate([xyz[0], xyz[1]], axis=0).T
        featc = jnp.concatenate([feat[0], feat[1]], axis=1)
        outs.append(_interp_stage(qpad, bpad, xyzT, featc, qb))
    xT = jnp.concatenate(outs, axis=0)
    h3p = 128
    W3p = jnp.pad(W3, ((0, h3p - W3.shape[0]), (0, 0)))
    W4p = jnp.pad(W4, ((0, 0), (0, h3p - W4.shape[1])))
    b4b = jnp.broadcast_to(b4.reshape(-1, 1), (W4.shape[0], 128))
    logitsT = _mlp(xT, W1, W2, W3p, W4p, b4b, 256)
    return logitsT.T
